# baseline (speedup 1.0000x reference)
import functools

import jax
import jax.numpy as jnp
from jax import lax
from jax.experimental import pallas as pl
from jax.experimental.pallas import tpu as pltpu
from jax.experimental.pallas import tpu_sc as plsc

T = 8192
D = 1024
NUM_CORES = 2
NUM_SUBCORES = 16
NUM_WORKERS = NUM_CORES * NUM_SUBCORES
ROWS_PER_WORKER = T // NUM_WORKERS
CHUNK = 64
NCHUNKS = ROWS_PER_WORKER // CHUNK


def _sc_copy_kernel():
    mesh = plsc.VectorSubcoreMesh(core_axis_name="c", subcore_axis_name="s")

    @functools.partial(
        pl.kernel,
        mesh=mesh,
        out_type=jax.ShapeDtypeStruct((T, D), jnp.float32),
        scratch_types=[
            pltpu.SemaphoreType.DMA,
        ],
    )
    def body(table_hbm, out_hbm, sem):
        wid = lax.axis_index("s") * NUM_CORES + lax.axis_index("c")
        base = wid * ROWS_PER_WORKER
        copies = []
        for i in range(NCHUNKS):
            sl = pl.ds(base + i * CHUNK, CHUNK)
            copies.append(
                pltpu.async_copy(table_hbm.at[sl, :], out_hbm.at[sl, :], sem))
        for c in copies:
            c.wait()

    return body


@jax.jit
def kernel(x, table):
    del x
    out = _sc_copy_kernel()(table)
    return out[None, :, :]


# speedup vs baseline: 24.8410x; 24.8410x over previous
import functools

import jax
import jax.numpy as jnp
from jax import lax
from jax.experimental import pallas as pl
from jax.experimental.pallas import tpu as pltpu
from jax.experimental.pallas import tpu_sc as plsc

T = 8192
D = 1024
NUM_CORES = 2
NUM_SUBCORES = 16
NUM_WORKERS = NUM_CORES * NUM_SUBCORES
ROWS_PER_WORKER = T // NUM_WORKERS
CHUNK = 32
NCHUNKS = ROWS_PER_WORKER // CHUNK
NBUF = 3


def _sc_copy_kernel():
    mesh = plsc.VectorSubcoreMesh(core_axis_name="c", subcore_axis_name="s")

    @functools.partial(
        pl.kernel,
        mesh=mesh,
        out_type=jax.ShapeDtypeStruct((T, D), jnp.float32),
        scratch_types=(
            [pltpu.VMEM((CHUNK, D), jnp.float32) for _ in range(NBUF)]
            + [pltpu.SemaphoreType.DMA for _ in range(2 * NBUF)]
        ),
    )
    def body(table_hbm, out_hbm, *scratch):
        bufs = scratch[:NBUF]
        isems = scratch[NBUF:2 * NBUF]
        osems = scratch[2 * NBUF:]
        wid = lax.axis_index("s") * NUM_CORES + lax.axis_index("c")
        base = wid * ROWS_PER_WORKER

        def start_in(i):
            return pltpu.async_copy(
                table_hbm.at[pl.ds(base + i * CHUNK, CHUNK), :],
                bufs[i % NBUF], isems[i % NBUF])

        def start_out(i):
            return pltpu.async_copy(
                bufs[i % NBUF],
                out_hbm.at[pl.ds(base + i * CHUNK, CHUNK), :],
                osems[i % NBUF])

        ins = {j: start_in(j) for j in range(NBUF - 1)}
        outs = {}
        for i in range(NCHUNKS):
            j = i + NBUF - 1
            if j < NCHUNKS:
                if j >= NBUF:
                    outs[j - NBUF].wait()
                ins[j] = start_in(j)
            ins[i].wait()
            outs[i] = start_out(i)
        for i in range(max(0, NCHUNKS - NBUF), NCHUNKS):
            outs[i].wait()

    return body


@jax.jit
def kernel(x, table):
    del x
    out = _sc_copy_kernel()(table)
    return out[None, :, :]


# speedup vs baseline: 24.8805x; 1.0016x over previous
import functools

import jax
import jax.numpy as jnp
from jax import lax
from jax.experimental import pallas as pl
from jax.experimental.pallas import tpu as pltpu
from jax.experimental.pallas import tpu_sc as plsc

T = 8192
D = 1024
NUM_CORES = 2
NUM_SUBCORES = 16
NUM_WORKERS = NUM_CORES * NUM_SUBCORES
ROWS_PER_WORKER = T // NUM_WORKERS
CHUNK = 16
NCHUNKS = ROWS_PER_WORKER // CHUNK
NBUF = 6


def _sc_copy_kernel():
    mesh = plsc.VectorSubcoreMesh(core_axis_name="c", subcore_axis_name="s")

    @functools.partial(
        pl.kernel,
        mesh=mesh,
        out_type=jax.ShapeDtypeStruct((T, D), jnp.float32),
        scratch_types=(
            [pltpu.VMEM((CHUNK, D), jnp.float32) for _ in range(NBUF)]
            + [pltpu.SemaphoreType.DMA for _ in range(2 * NBUF)]
        ),
    )
    def body(table_hbm, out_hbm, *scratch):
        bufs = scratch[:NBUF]
        isems = scratch[NBUF:2 * NBUF]
        osems = scratch[2 * NBUF:]
        wid = lax.axis_index("s") * NUM_CORES + lax.axis_index("c")
        base = wid * ROWS_PER_WORKER

        def start_in(i):
            return pltpu.async_copy(
                table_hbm.at[pl.ds(base + i * CHUNK, CHUNK), :],
                bufs[i % NBUF], isems[i % NBUF])

        def start_out(i):
            return pltpu.async_copy(
                bufs[i % NBUF],
                out_hbm.at[pl.ds(base + i * CHUNK, CHUNK), :],
                osems[i % NBUF])

        ins = {j: start_in(j) for j in range(NBUF - 1)}
        outs = {}
        for i in range(NCHUNKS):
            j = i + NBUF - 1
            if j < NCHUNKS:
                if j >= NBUF:
                    outs[j - NBUF].wait()
                ins[j] = start_in(j)
            ins[i].wait()
            outs[i] = start_out(i)
        for i in range(max(0, NCHUNKS - NBUF), NCHUNKS):
            outs[i].wait()

    return body


@jax.jit
def kernel(x, table):
    del x
    out = _sc_copy_kernel()(table)
    return out[None, :, :]


# speedup vs baseline: 42.2028x; 1.6962x over previous
import functools

import jax
import jax.numpy as jnp
from jax.experimental import pallas as pl
from jax.experimental.pallas import tpu as pltpu

T = 8192
D = 1024
BLOCK = 512


def _body(t_ref, o_ref):
    o_ref[...] = t_ref[...]


@jax.jit
def kernel(x, table):
    del x
    out = pl.pallas_call(
        _body,
        grid=(T // BLOCK,),
        in_specs=[pl.BlockSpec((BLOCK, D), lambda i: (i, 0))],
        out_specs=pl.BlockSpec((BLOCK, D), lambda i: (i, 0)),
        out_shape=jax.ShapeDtypeStruct((T, D), jnp.float32),
    )(table)
    return out[None, :, :]
